# Initial kernel scaffold; baseline (speedup 1.0000x reference)
import jax
import jax.numpy as jnp
from jax.experimental import pallas as pl


def kernel(x):
    raise NotImplementedError("write your pallas kernel here")


# speedup vs baseline: 7070.0917x; 7070.0917x over previous
import functools

import jax
import jax.numpy as jnp
from jax import lax
from jax.experimental import pallas as pl
from jax.experimental.pallas import tpu as pltpu
from jax.experimental.pallas import tpu_sc as plsc

_N = 4 * 4096 * 4096
_NB = 1024
_INV_D = 997.0 / 2.0
_THRESH = 0.1

_NC = 2
_NS = 16
_NW = _NC * _NS
_CHUNK = _N // _NW
_BLK = 16384
_NBLK = _CHUNK // _BLK
_SL = _BLK // 16

_R = 16384
_C = 4096
_BR = 256


def _sc_hist(packed):
    mesh = plsc.VectorSubcoreMesh(core_axis_name="c", subcore_axis_name="s")

    @functools.partial(
        pl.kernel,
        out_type=jax.ShapeDtypeStruct((_NW, 2, _NB), jnp.float32),
        mesh=mesh,
        compiler_params=pltpu.CompilerParams(
            needs_layout_passes=False, use_tc_tiling_on_sc=False),
        scratch_types=[
            pltpu.VMEM((2, _BLK), jnp.int32),
            pltpu.VMEM((_NB, 16), jnp.float32),
            pltpu.VMEM((_NB, 16), jnp.float32),
            pltpu.VMEM((2, _NB), jnp.float32),
            pltpu.SemaphoreType.DMA,
            pltpu.SemaphoreType.DMA,
        ],
    )
    def k(p_hbm, out_hbm, buf, ha, hb, red, sem0, sem1):
        wid = lax.axis_index("s") * _NC + lax.axis_index("c")
        base = wid * _CHUNK

        zeros16 = jnp.zeros((16,), jnp.float32)

        @pl.loop(0, _NB)
        def _zero(b):
            ha[b, :] = zeros16
            hb[b, :] = zeros16

        ones16 = jnp.ones((16,), jnp.float32)
        lanes = lax.iota(jnp.int32, 16)

        def start(blk, slot, sem):
            pltpu.async_copy(
                p_hbm.at[pl.ds(base + blk * _BLK, _BLK)], buf.at[slot], sem)

        def wait(blk, slot, sem):
            pltpu.make_async_copy(
                p_hbm.at[pl.ds(base + blk * _BLK, _BLK)], buf.at[slot],
                sem).wait()

        def process(slot):
            @pl.loop(0, _SL, unroll=8)
            def _body(s):
                w = buf[slot, pl.ds(s * 16, 16)]
                ba = jnp.bitwise_and(w, 1023)
                bb = lax.shift_right_logical(w, 10)
                plsc.addupdate_scatter(ha, [ba, lanes], ones16)
                plsc.addupdate_scatter(hb, [bb, lanes], ones16)

        start(0, 0, sem0)

        @pl.loop(0, _NBLK // 2)
        def _outer(i):
            blk0 = i * 2
            start(blk0 + 1, 1, sem1)
            wait(blk0, 0, sem0)
            process(0)

            @pl.when(i < _NBLK // 2 - 1)
            def _():
                start(blk0 + 2, 0, sem0)

            wait(blk0 + 1, 1, sem1)
            process(1)

        @pl.loop(0, _NB // 16)
        def _reduce(j):
            bins16 = j * 16 + lanes
            acc_a = zeros16
            acc_b = zeros16
            for l in range(16):
                li = jnp.full((16,), l, jnp.int32)
                acc_a = acc_a + plsc.load_gather(ha, [bins16, li])
                acc_b = acc_b + plsc.load_gather(hb, [bins16, li])
            red[0, pl.ds(j * 16, 16)] = acc_a
            red[1, pl.ds(j * 16, 16)] = acc_b

        pltpu.sync_copy(red, out_hbm.at[wid])

    return k(packed)


def _tc_stream(x2d):
    def body(x_ref, o_ref, p_ref, nz_ref):
        @pl.when(pl.program_id(0) == 0)
        def _():
            nz_ref[0, 0] = 0

        v = jax.nn.silu(x_ref[...])
        av = jnp.abs(v)
        m = av < _THRESH
        o_ref[...] = jnp.where(m, 0.0, v)
        nz_ref[0, 0] += jnp.sum(m.astype(jnp.int32))

        ta = (v + 1.0) * _INV_D
        tb = (av + 1.0) * _INV_D
        ba = jnp.minimum(ta.astype(jnp.int32) + 1, 998)
        bb = jnp.minimum(tb.astype(jnp.int32) + 1, 998)
        p_ref[...] = ba + bb * 1024

    return pl.pallas_call(
        body,
        grid=(_R // _BR,),
        in_specs=[pl.BlockSpec((_BR, _C), lambda i: (i, 0))],
        out_specs=[
            pl.BlockSpec((_BR, _C), lambda i: (i, 0)),
            pl.BlockSpec((_BR, _C), lambda i: (i, 0)),
            pl.BlockSpec(memory_space=pltpu.SMEM),
        ],
        out_shape=[
            jax.ShapeDtypeStruct((_R, _C), jnp.float32),
            jax.ShapeDtypeStruct((_R, _C), jnp.int32),
            jax.ShapeDtypeStruct((1, 1), jnp.int32),
        ],
    )(x2d)


def _tc_reduce(parts):
    def body(p_ref, o_ref):
        o_ref[...] = jnp.sum(p_ref[...], axis=0)

    return pl.pallas_call(
        body,
        out_shape=jax.ShapeDtypeStruct((2, _NB), jnp.float32),
    )(parts)


def kernel(x):
    out2d, packed, nz = _tc_stream(x.reshape(_R, _C))
    parts = _sc_hist(packed.reshape(-1))
    hists = _tc_reduce(parts)

    return (
        out2d.reshape(x.shape),
        hists[0, :999],
        hists[1, :999],
        nz[0, 0],
        jnp.asarray(_N),
    )


# speedup vs baseline: 11609.1797x; 1.6420x over previous
import functools

import jax
import jax.numpy as jnp
from jax import lax
from jax.experimental import pallas as pl
from jax.experimental.pallas import tpu as pltpu
from jax.experimental.pallas import tpu_sc as plsc

_N = 4 * 4096 * 4096
_NB = 1024
_INV_D = 997.0 / 2.0
_THRESH = 0.1

_NC = 2
_NS = 16
_NW = _NC * _NS
_CHUNK = _N // _NW
_BLK = 16384
_NBLK = _CHUNK // _BLK
_SL = _BLK // 16

_R = 16384
_C = 4096
_BR = 256


def _sc_hist(packed):
    mesh = plsc.VectorSubcoreMesh(core_axis_name="c", subcore_axis_name="s")

    @functools.partial(
        pl.kernel,
        out_type=jax.ShapeDtypeStruct((_NW, 2, _NB), jnp.float32),
        mesh=mesh,
        compiler_params=pltpu.CompilerParams(
            needs_layout_passes=False, use_tc_tiling_on_sc=False),
        scratch_types=[
            pltpu.VMEM((2, _BLK), jnp.int32),
            pltpu.VMEM((_NB * 16,), jnp.float32),
            pltpu.VMEM((_NB * 16,), jnp.float32),
            pltpu.VMEM((2, _NB), jnp.float32),
            pltpu.SemaphoreType.DMA,
            pltpu.SemaphoreType.DMA,
        ],
    )
    def k(p_hbm, out_hbm, buf, ha, hb, red, sem0, sem1):
        wid = lax.axis_index("s") * _NC + lax.axis_index("c")
        base = wid * _CHUNK

        zeros16 = jnp.zeros((16,), jnp.float32)

        @pl.loop(0, _NB)
        def _zero(b):
            ha[pl.ds(b * 16, 16)] = zeros16
            hb[pl.ds(b * 16, 16)] = zeros16

        ones16 = jnp.ones((16,), jnp.float32)
        lanes = lax.iota(jnp.int32, 16)

        def start(blk, slot, sem):
            pltpu.async_copy(
                p_hbm.at[pl.ds(base + blk * _BLK, _BLK)], buf.at[slot], sem)

        def wait(blk, slot, sem):
            pltpu.make_async_copy(
                p_hbm.at[pl.ds(base + blk * _BLK, _BLK)], buf.at[slot],
                sem).wait()

        def process(slot):
            @pl.loop(0, _SL // 4, unroll=2)
            def _body(q):
                ws = [buf[slot, pl.ds((q * 4 + i) * 16, 16)]
                      for i in range(4)]
                aas = [jnp.bitwise_and(w, 0xFFFF) for w in ws]
                abs_ = [lax.shift_right_logical(w, 16) for w in ws]
                for i in range(4):
                    plsc.addupdate_scatter(ha, [aas[i]], ones16)
                    plsc.addupdate_scatter(hb, [abs_[i]], ones16)

        start(0, 0, sem0)

        @pl.loop(0, _NBLK // 2)
        def _outer(i):
            blk0 = i * 2
            start(blk0 + 1, 1, sem1)
            wait(blk0, 0, sem0)
            process(0)

            @pl.when(i < _NBLK // 2 - 1)
            def _():
                start(blk0 + 2, 0, sem0)

            wait(blk0 + 1, 1, sem1)
            process(1)

        @pl.loop(0, _NB // 16)
        def _reduce(j):
            flat16 = (j * 16 + lanes) * 16
            acc_a = zeros16
            acc_b = zeros16
            for l in range(16):
                acc_a = acc_a + plsc.load_gather(ha, [flat16 + l])
                acc_b = acc_b + plsc.load_gather(hb, [flat16 + l])
            red[0, pl.ds(j * 16, 16)] = acc_a
            red[1, pl.ds(j * 16, 16)] = acc_b

        pltpu.sync_copy(red, out_hbm.at[wid])

    return k(packed)


def _tc_stream(x2d):
    def body(x_ref, o_ref, p_ref, nz_ref):
        @pl.when(pl.program_id(0) == 0)
        def _():
            nz_ref[0, 0] = 0

        v = jax.nn.silu(x_ref[...])
        av = jnp.abs(v)
        m = av < _THRESH
        o_ref[...] = jnp.where(m, 0.0, v)
        nz_ref[0, 0] += jnp.sum(m.astype(jnp.int32))

        ta = (v + 1.0) * _INV_D
        tb = (av + 1.0) * _INV_D
        ba = jnp.minimum(ta.astype(jnp.int32) + 1, 998)
        bb = jnp.minimum(tb.astype(jnp.int32) + 1, 998)
        lane = jnp.bitwise_and(
            lax.broadcasted_iota(jnp.int32, (_BR, _C), 1), 15)
        pa = jnp.bitwise_or(lax.shift_left(ba, 4), lane)
        pb = jnp.bitwise_or(lax.shift_left(bb, 4), lane)
        p_ref[...] = jnp.bitwise_or(pa, lax.shift_left(pb, 16))

    return pl.pallas_call(
        body,
        grid=(_R // _BR,),
        in_specs=[pl.BlockSpec((_BR, _C), lambda i: (i, 0))],
        out_specs=[
            pl.BlockSpec((_BR, _C), lambda i: (i, 0)),
            pl.BlockSpec((_BR, _C), lambda i: (i, 0)),
            pl.BlockSpec(memory_space=pltpu.SMEM),
        ],
        out_shape=[
            jax.ShapeDtypeStruct((_R, _C), jnp.float32),
            jax.ShapeDtypeStruct((_R, _C), jnp.int32),
            jax.ShapeDtypeStruct((1, 1), jnp.int32),
        ],
    )(x2d)


def _tc_reduce(parts):
    def body(p_ref, o_ref):
        o_ref[...] = jnp.sum(p_ref[...], axis=0)

    return pl.pallas_call(
        body,
        out_shape=jax.ShapeDtypeStruct((2, _NB), jnp.float32),
    )(parts)


def kernel(x):
    out2d, packed, nz = _tc_stream(x.reshape(_R, _C))
    parts = _sc_hist(packed.reshape(-1))
    hists = _tc_reduce(parts)

    return (
        out2d.reshape(x.shape),
        hists[0, :999],
        hists[1, :999],
        nz[0, 0],
        jnp.asarray(_N),
    )
